# baseline (speedup 1.0000x reference)
import functools

import jax
import jax.numpy as jnp
from jax import lax
from jax.experimental import pallas as pl
from jax.experimental.pallas import tpu as pltpu
from jax.experimental.pallas import tpu_sc as plsc

GRID = 8
NB = 32.0
NCELLS = GRID * GRID
INV_CELL = 1.0 / (2.0 * NB / GRID)

NC, NS, L = 2, 16, 16
NW = NC * NS

def _lane_bcast(x, idx):
    dnums = lax.GatherDimensionNumbers(
        offset_dims=(), collapsed_slice_dims=(0,), start_index_map=(0,))
    return lax.gather(x, idx[:, None], dnums, slice_sizes=(1,),
                      mode=lax.GatherScatterMode.PROMISE_IN_BOUNDS)


NI = 8
CH = 128
GR = (NCELLS + 1) * 128


def _sc_bin_body(sx_hbm, sy_hbm, hid_hbm, out_hbm, sxv, syv, hbuf,
                 grids, *, a, h, aw):
    wid = lax.axis_index("s") * NC + lax.axis_index("c")
    pltpu.sync_copy(sx_hbm, sxv)
    pltpu.sync_copy(sy_hbm, syv)
    hiota = lax.iota(jnp.int32, L)
    nch = a // CH
    npg = CH // L

    def ib_body(ib, _):
        def zero_body(k, _):
            grids[pl.ds(k * L, L)] = jnp.zeros((L,), jnp.float32)
            return 0
        lax.fori_loop(0, NI * GR // L, zero_body, 0)

        i_base = wid * aw + ib * NI

        def ch_body(ch, _):
            pltpu.sync_copy(hid_hbm.at[pl.ds(ch * CH, CH), :], hbuf)

            def il_body(il, _):
                i = i_base + il
                lane = hiota * 0 + (i % L)
                xg = sxv[pl.ds((i // L) * L, L)]
                yg = syv[pl.ds((i // L) * L, L)]
                xi = _lane_bcast(xg, lane)
                yi = _lane_bcast(yg, lane)
                gbase = hiota + il * GR

                def jg_body(jg, _):
                    j0 = ch * CH + jg * L
                    xj = sxv[pl.ds(j0, L)]
                    yj = syv[pl.ds(j0, L)]
                    rx = xj - xi
                    ry = yj - yi
                    col = ((rx + NB) * INV_CELL).astype(jnp.int32)
                    row = ((ry + NB) * INV_CELL).astype(jnp.int32)
                    ok = (jnp.abs(rx) < NB) & (jnp.abs(ry) < NB)
                    ok = ok & ((hiota + j0) != i)
                    cell = (row << 3) + col
                    cell = jnp.where(ok, cell, NCELLS)
                    cell128 = cell << 7

                    def p_body(p, _):
                        cb = _lane_bcast(cell128, hiota * 0 + p)
                        idx0 = cb + gbase
                        jrow = jg * L + p
                        for s in range(h // L):
                            val = hbuf[jrow, pl.ds(s * L, L)]
                            plsc.addupdate_scatter(grids, [idx0 + s * L], val)
                        return 0
                    lax.fori_loop(0, L, p_body, 0)
                    return 0
                lax.fori_loop(0, npg, jg_body, 0)
                return 0
            lax.fori_loop(0, NI, il_body, 0)
            return 0
        lax.fori_loop(0, nch, ch_body, 0)

        def drain_body(il, _):
            pltpu.sync_copy(grids.at[pl.ds(il * GR, NCELLS * h)],
                            out_hbm.at[i_base + il])
            return 0
        lax.fori_loop(0, NI, drain_body, 0)
        return 0
    lax.fori_loop(0, aw // NI, ib_body, 0)


def _sc_bin(sx, sy, hidden):
    a, h = hidden.shape
    aw = a // NW
    mesh = plsc.VectorSubcoreMesh(core_axis_name="c", subcore_axis_name="s")
    return pl.kernel(
        functools.partial(_sc_bin_body, a=a, h=h, aw=aw),
        out_type=jax.ShapeDtypeStruct((a, NCELLS * h), jnp.float32),
        mesh=mesh,
        compiler_params=pltpu.CompilerParams(needs_layout_passes=False),
        scratch_types=[
            pltpu.VMEM((a,), jnp.float32),
            pltpu.VMEM((a,), jnp.float32),
            pltpu.VMEM((CH, h), jnp.float32),
            pltpu.VMEM((NI * GR,), jnp.float32),
        ],
    )(sx, sy, hidden)


def _mm_body(g_ref, wt_ref, b2_ref, mc_ref, out_ref):
    gb = g_ref[...].astype(jnp.bfloat16)
    acc = jnp.dot(gb, wt_ref[...], preferred_element_type=jnp.float32)
    out_ref[...] = (acc + b2_ref[...]) * mc_ref[...]


def _tc_matmul(grids, wt, b2, maskc):
    a = grids.shape[0]
    h = wt.shape[1]
    bi = 256 if a % 256 == 0 else a
    return pl.pallas_call(
        _mm_body,
        grid=(a // bi,),
        in_specs=[
            pl.BlockSpec((bi, NCELLS * h), lambda i: (i, 0)),
            pl.BlockSpec((NCELLS * h, h), lambda i: (0, 0)),
            pl.BlockSpec((1, h), lambda i: (0, 0)),
            pl.BlockSpec((bi, 1), lambda i: (i, 0)),
        ],
        out_specs=pl.BlockSpec((bi, h), lambda i: (i, 0)),
        out_shape=jax.ShapeDtypeStruct((a, h), jnp.float32),
    )(grids, wt, b2, maskc)


def kernel(hidden, pos, mask, W, b):
    a, h = hidden.shape
    mask_f = mask.astype(jnp.float32)
    big = jnp.float32(1e30)
    sx = jnp.where(mask, pos[:, 0], big)
    sy = jnp.where(mask, pos[:, 1], big)
    grids = _sc_bin(sx, sy, hidden)
    wt = W.T.astype(jnp.bfloat16)
    b2 = b.reshape(1, h)
    maskc = mask_f.reshape(a, 1)
    return _tc_matmul(grids, wt, b2, maskc)


# speedup vs baseline: 2.3260x; 2.3260x over previous
import functools

import jax
import jax.numpy as jnp
from jax import lax
from jax.experimental import pallas as pl
from jax.experimental.pallas import tpu as pltpu
from jax.experimental.pallas import tpu_sc as plsc

GRID = 8
NB = 32.0
NCELLS = GRID * GRID
INV_CELL = 1.0 / (2.0 * NB / GRID)

NC, NS, L = 2, 16, 16
NW = NC * NS

def _lane_bcast(x, idx):
    dnums = lax.GatherDimensionNumbers(
        offset_dims=(), collapsed_slice_dims=(0,), start_index_map=(0,))
    return lax.gather(x, idx[:, None], dnums, slice_sizes=(1,),
                      mode=lax.GatherScatterMode.PROMISE_IN_BOUNDS)


NI = 8
CH = 128
GR = (NCELLS + 1) * 128


def _sc_bin_body(sx_hbm, sy_hbm, hid_hbm, out_hbm, sxv, syv, hbuf,
                 grids, *, a, h, aw):
    wid = lax.axis_index("s") * NC + lax.axis_index("c")
    pltpu.sync_copy(sx_hbm, sxv)
    pltpu.sync_copy(sy_hbm, syv)
    hiota = lax.iota(jnp.int32, L)
    nch = a // CH
    npg = CH // L

    def ib_body(ib, _):
        def zero_body(k, _):
            grids[pl.ds(k * L, L)] = jnp.zeros((L,), jnp.float32)
            return 0
        lax.fori_loop(0, NI * GR // L, zero_body, 0)

        i_base = wid * aw + ib * NI

        def ch_body(ch, _):
            pltpu.sync_copy(hid_hbm.at[pl.ds(ch * CH, CH), :], hbuf)

            def il_body(il, _):
                i = i_base + il
                lane = hiota * 0 + (i % L)
                xg = sxv[pl.ds((i // L) * L, L)]
                yg = syv[pl.ds((i // L) * L, L)]
                xi = _lane_bcast(xg, lane)
                yi = _lane_bcast(yg, lane)
                gbase = hiota + il * GR

                def jg_body(jg, _):
                    j0 = ch * CH + jg * L
                    xj = sxv[pl.ds(j0, L)]
                    yj = syv[pl.ds(j0, L)]
                    rx = xj - xi
                    ry = yj - yi
                    col = ((rx + NB) * INV_CELL).astype(jnp.int32)
                    row = ((ry + NB) * INV_CELL).astype(jnp.int32)
                    ok = (jnp.abs(rx) < NB) & (jnp.abs(ry) < NB)
                    ok = ok & ((hiota + j0) != i)
                    cell = (row << 3) + col
                    cell = jnp.where(ok, cell, NCELLS)
                    cell128 = cell << 7

                    @plsc.parallel_loop(0, L, 1, unroll=2)
                    def p_body(p):
                        cb = _lane_bcast(cell128, hiota * 0 + p)
                        idx0 = cb + gbase
                        jrow = jg * L + p
                        for s in range(h // L):
                            val = hbuf[jrow, pl.ds(s * L, L)]
                            plsc.addupdate_scatter(grids, [idx0 + s * L], val)
                    return 0
                lax.fori_loop(0, npg, jg_body, 0)
                return 0
            lax.fori_loop(0, NI, il_body, 0)
            return 0
        lax.fori_loop(0, nch, ch_body, 0)

        def drain_body(il, _):
            pltpu.sync_copy(grids.at[pl.ds(il * GR, NCELLS * h)],
                            out_hbm.at[i_base + il])
            return 0
        lax.fori_loop(0, NI, drain_body, 0)
        return 0
    lax.fori_loop(0, aw // NI, ib_body, 0)


def _sc_bin(sx, sy, hidden):
    a, h = hidden.shape
    aw = a // NW
    mesh = plsc.VectorSubcoreMesh(core_axis_name="c", subcore_axis_name="s")
    return pl.kernel(
        functools.partial(_sc_bin_body, a=a, h=h, aw=aw),
        out_type=jax.ShapeDtypeStruct((a, NCELLS * h), jnp.float32),
        mesh=mesh,
        compiler_params=pltpu.CompilerParams(needs_layout_passes=False),
        scratch_types=[
            pltpu.VMEM((a,), jnp.float32),
            pltpu.VMEM((a,), jnp.float32),
            pltpu.VMEM((CH, h), jnp.float32),
            pltpu.VMEM((NI * GR,), jnp.float32),
        ],
    )(sx, sy, hidden)


def _mm_body(g_ref, wt_ref, b2_ref, mc_ref, out_ref):
    gb = g_ref[...].astype(jnp.bfloat16)
    acc = jnp.dot(gb, wt_ref[...], preferred_element_type=jnp.float32)
    out_ref[...] = (acc + b2_ref[...]) * mc_ref[...]


def _tc_matmul(grids, wt, b2, maskc):
    a = grids.shape[0]
    h = wt.shape[1]
    bi = 256 if a % 256 == 0 else a
    return pl.pallas_call(
        _mm_body,
        grid=(a // bi,),
        in_specs=[
            pl.BlockSpec((bi, NCELLS * h), lambda i: (i, 0)),
            pl.BlockSpec((NCELLS * h, h), lambda i: (0, 0)),
            pl.BlockSpec((1, h), lambda i: (0, 0)),
            pl.BlockSpec((bi, 1), lambda i: (i, 0)),
        ],
        out_specs=pl.BlockSpec((bi, h), lambda i: (i, 0)),
        out_shape=jax.ShapeDtypeStruct((a, h), jnp.float32),
    )(grids, wt, b2, maskc)


def kernel(hidden, pos, mask, W, b):
    a, h = hidden.shape
    mask_f = mask.astype(jnp.float32)
    big = jnp.float32(1e30)
    sx = jnp.where(mask, pos[:, 0], big)
    sy = jnp.where(mask, pos[:, 1], big)
    grids = _sc_bin(sx, sy, hidden)
    wt = W.T.astype(jnp.bfloat16)
    b2 = b.reshape(1, h)
    maskc = mask_f.reshape(a, 1)
    return _tc_matmul(grids, wt, b2, maskc)


# speedup vs baseline: 2.4313x; 1.0453x over previous
import functools

import jax
import jax.numpy as jnp
from jax import lax
from jax.experimental import pallas as pl
from jax.experimental.pallas import tpu as pltpu
from jax.experimental.pallas import tpu_sc as plsc

GRID = 8
NB = 32.0
NCELLS = GRID * GRID
INV_CELL = 1.0 / (2.0 * NB / GRID)

NC, NS, L = 2, 16, 16
NW = NC * NS

def _lane_bcast(x, idx):
    dnums = lax.GatherDimensionNumbers(
        offset_dims=(), collapsed_slice_dims=(0,), start_index_map=(0,))
    return lax.gather(x, idx[:, None], dnums, slice_sizes=(1,),
                      mode=lax.GatherScatterMode.PROMISE_IN_BOUNDS)


NI = 8
CH = 128
GR = (NCELLS + 1) * 128


def _sc_bin_body(sx_hbm, sy_hbm, hid_hbm, out_hbm, sxv, syv, hbuf,
                 grids, *, a, h, aw):
    wid = lax.axis_index("s") * NC + lax.axis_index("c")
    pltpu.sync_copy(sx_hbm, sxv)
    pltpu.sync_copy(sy_hbm, syv)
    hiota = lax.iota(jnp.int32, L)
    nch = a // CH
    npg = CH // L

    def ib_body(ib, _):
        def zero_body(k, _):
            grids[pl.ds(k * L, L)] = jnp.zeros((L,), jnp.float32)
            return 0
        lax.fori_loop(0, NI * GR // L, zero_body, 0)

        i_base = wid * aw + ib * NI

        def ch_body(ch, _):
            pltpu.sync_copy(hid_hbm.at[pl.ds(ch * CH, CH), :], hbuf)

            def il_body(il, _):
                i = i_base + il
                lane = hiota * 0 + (i % L)
                xg = sxv[pl.ds((i // L) * L, L)]
                yg = syv[pl.ds((i // L) * L, L)]
                xi = _lane_bcast(xg, lane)
                yi = _lane_bcast(yg, lane)
                gbase = hiota + il * GR

                def jg_body(jg, _):
                    j0 = ch * CH + jg * L
                    xj = sxv[pl.ds(j0, L)]
                    yj = syv[pl.ds(j0, L)]
                    rx = xj - xi
                    ry = yj - yi
                    col = ((rx + NB) * INV_CELL).astype(jnp.int32)
                    row = ((ry + NB) * INV_CELL).astype(jnp.int32)
                    ok = (jnp.abs(rx) < NB) & (jnp.abs(ry) < NB)
                    ok = ok & ((hiota + j0) != i)
                    cell = (row << 3) + col
                    cell = jnp.where(ok, cell, NCELLS)
                    cell128 = cell << 7

                    @plsc.parallel_loop(0, L, 1, unroll=4)
                    def p_body(p):
                        cb = _lane_bcast(cell128, hiota * 0 + p)
                        idx0 = cb + gbase
                        jrow = jg * L + p
                        for s in range(h // L):
                            val = hbuf[jrow, pl.ds(s * L, L)]
                            plsc.addupdate_scatter(grids, [idx0 + s * L], val)
                    return 0
                lax.fori_loop(0, npg, jg_body, 0)
                return 0
            lax.fori_loop(0, NI, il_body, 0)
            return 0
        lax.fori_loop(0, nch, ch_body, 0)

        def drain_body(il, _):
            pltpu.sync_copy(grids.at[pl.ds(il * GR, NCELLS * h)],
                            out_hbm.at[i_base + il])
            return 0
        lax.fori_loop(0, NI, drain_body, 0)
        return 0
    lax.fori_loop(0, aw // NI, ib_body, 0)


def _sc_bin(sx, sy, hidden):
    a, h = hidden.shape
    aw = a // NW
    mesh = plsc.VectorSubcoreMesh(core_axis_name="c", subcore_axis_name="s")
    return pl.kernel(
        functools.partial(_sc_bin_body, a=a, h=h, aw=aw),
        out_type=jax.ShapeDtypeStruct((a, NCELLS * h), jnp.float32),
        mesh=mesh,
        compiler_params=pltpu.CompilerParams(needs_layout_passes=False),
        scratch_types=[
            pltpu.VMEM((a,), jnp.float32),
            pltpu.VMEM((a,), jnp.float32),
            pltpu.VMEM((CH, h), jnp.float32),
            pltpu.VMEM((NI * GR,), jnp.float32),
        ],
    )(sx, sy, hidden)


def _mm_body(g_ref, wt_ref, b2_ref, mc_ref, out_ref):
    gb = g_ref[...].astype(jnp.bfloat16)
    acc = jnp.dot(gb, wt_ref[...], preferred_element_type=jnp.float32)
    out_ref[...] = (acc + b2_ref[...]) * mc_ref[...]


def _tc_matmul(grids, wt, b2, maskc):
    a = grids.shape[0]
    h = wt.shape[1]
    bi = 256 if a % 256 == 0 else a
    return pl.pallas_call(
        _mm_body,
        grid=(a // bi,),
        in_specs=[
            pl.BlockSpec((bi, NCELLS * h), lambda i: (i, 0)),
            pl.BlockSpec((NCELLS * h, h), lambda i: (0, 0)),
            pl.BlockSpec((1, h), lambda i: (0, 0)),
            pl.BlockSpec((bi, 1), lambda i: (i, 0)),
        ],
        out_specs=pl.BlockSpec((bi, h), lambda i: (i, 0)),
        out_shape=jax.ShapeDtypeStruct((a, h), jnp.float32),
    )(grids, wt, b2, maskc)


def kernel(hidden, pos, mask, W, b):
    a, h = hidden.shape
    mask_f = mask.astype(jnp.float32)
    big = jnp.float32(1e30)
    sx = jnp.where(mask, pos[:, 0], big)
    sy = jnp.where(mask, pos[:, 1], big)
    grids = _sc_bin(sx, sy, hidden)
    wt = W.T.astype(jnp.bfloat16)
    b2 = b.reshape(1, h)
    maskc = mask_f.reshape(a, 1)
    return _tc_matmul(grids, wt, b2, maskc)
